# Initial kernel scaffold; baseline (speedup 1.0000x reference)
import jax
import jax.numpy as jnp
from jax.experimental import pallas as pl


def kernel(x_circ, x_mir, x_dis, ei_c_int_m, ei_m_int_d, ei_c_as_d, ei_m_rev_c, ei_d_rev_m, ei_d_rva_c, ei_c_gip_c, ei_m_gip_m, W1l, W1r, b1, W2l, W2r, b2, W3l, W3r, b3):
    raise NotImplementedError("write your pallas kernel here")


# speedup vs baseline: 2.0873x; 2.0873x over previous
import functools

import jax
import jax.numpy as jnp
from jax import lax
from jax.experimental import pallas as pl
from jax.experimental.pallas import tpu as pltpu
from jax.experimental.pallas import tpu_sc as plsc

N = 40000
DM = 64
HALF = N // 2
NS = 16
CHUNK = 128
BCH = 256
ZB = 1280
ACC_ROWS = ZB * NS
FLUSH = HALF // NS
FCH = 125
BLK = 2000


def _sc_mesh():
    return plsc.VectorSubcoreMesh(core_axis_name="c", subcore_axis_name="s")


def _make_segsum(e_pad):
    chunks = e_pad // (NS * BCH)
    nb = BCH // CHUNK

    def _buf_types():
        return [pltpu.VMEM((BCH,), jnp.int32),
                pltpu.VMEM((BCH,), jnp.int32),
                [pltpu.VMEM((CHUNK,), jnp.int32) for _ in range(nb)],
                pltpu.VMEM((BCH, DM), jnp.float32),
                pltpu.SemaphoreType.DMA]

    @functools.partial(
        pl.kernel,
        out_type=jax.ShapeDtypeStruct((N, DM), jnp.float32),
        mesh=_sc_mesh(),
        compiler_params=pltpu.CompilerParams(use_tc_tiling_on_sc=False),
        scratch_types=[
            pltpu.VMEM_SHARED((ACC_ROWS, DM), jnp.float32),
            pltpu.VMEM((CHUNK, DM), jnp.float32),
            _buf_types(),
            _buf_types(),
        ],
    )
    def seg(p_hbm, src_hbm, dst_hbm, zeros_hbm, s_hbm,
            acc_sh, zbuf, buf_a, buf_b):
        cid = lax.axis_index("c")
        sid = lax.axis_index("s")
        rebase = cid * HALF
        pltpu.sync_copy(zeros_hbm, zbuf)
        zoff = pl.multiple_of(sid * ZB, 8)

        def zloop(z, carry):
            pltpu.sync_copy(zbuf, acc_sh.at[pl.ds(zoff + z * CHUNK, CHUNK)])
            return carry

        lax.fori_loop(0, ZB // CHUNK, zloop, 0)
        plsc.subcore_barrier()
        tile_base = sid * (chunks * BCH)

        def issue(buf, c):
            src_v, dst_v, idxs, rows_v, sem = buf
            eb = pl.multiple_of(tile_base + c * BCH, 8)
            pltpu.sync_copy(src_hbm.at[pl.ds(eb, BCH)], src_v)
            pltpu.sync_copy(dst_hbm.at[pl.ds(eb, BCH)], dst_v)
            for j in range(nb):
                pltpu.async_copy(p_hbm.at[src_v.at[pl.ds(j * CHUNK, CHUNK)]],
                                 rows_v.at[pl.ds(j * CHUNK, CHUNK)], sem)
            for j in range(nb):
                for i in range(CHUNK // 16):
                    d = dst_v[pl.ds(j * CHUNK + i * 16, 16)]
                    loc = d - rebase
                    ok = (loc >= 0) & (loc < HALF)
                    idxs[j][pl.ds(i * 16, 16)] = jnp.where(ok, loc, HALF)

        def drain_scatter(buf):
            src_v, dst_v, idxs, rows_v, sem = buf
            for j in range(nb):
                pltpu.make_async_copy(
                    p_hbm.at[src_v.at[pl.ds(j * CHUNK, CHUNK)]],
                    rows_v.at[pl.ds(j * CHUNK, CHUNK)], sem).wait()
            for j in range(nb):
                pltpu.sync_copy(rows_v.at[pl.ds(j * CHUNK, CHUNK)],
                                acc_sh.at[idxs[j]], add=True)

        issue(buf_a, 0)

        def body(t, carry):
            issue(buf_b, 2 * t + 1)
            drain_scatter(buf_a)

            @pl.when(t < chunks // 2 - 1)
            def _():
                issue(buf_a, 2 * t + 2)

            drain_scatter(buf_b)
            return carry

        lax.fori_loop(0, chunks // 2, body, 0)
        plsc.subcore_barrier()
        foff = sid * FLUSH

        def floop(f, carry):
            ro = foff + f * FCH
            pltpu.sync_copy(acc_sh.at[pl.ds(ro, FCH)], zbuf.at[pl.ds(0, FCH)])
            pltpu.sync_copy(zbuf.at[pl.ds(0, FCH)], s_hbm.at[pl.ds(rebase + ro, FCH)])
            return carry

        lax.fori_loop(0, FLUSH // FCH, floop, 0)

    return seg


def _make_count(e_pad):
    chunks = e_pad // (NS * CHUNK)

    @functools.partial(
        pl.kernel,
        out_type=jax.ShapeDtypeStruct((N, DM), jnp.float32),
        mesh=_sc_mesh(),
        compiler_params=pltpu.CompilerParams(use_tc_tiling_on_sc=False),
        scratch_types=[
            pltpu.VMEM_SHARED((ACC_ROWS, DM), jnp.float32),
            pltpu.VMEM((CHUNK,), jnp.int32),
            pltpu.VMEM((CHUNK, DM), jnp.float32),
            pltpu.VMEM((CHUNK, DM), jnp.float32),
        ],
    )
    def cnt(dst_hbm, zeros_hbm, ones_hbm, c_hbm, acc_sh, idx_v, ones_v, zbuf):
        cid = lax.axis_index("c")
        sid = lax.axis_index("s")
        rebase = cid * HALF
        pltpu.sync_copy(zeros_hbm, zbuf)
        pltpu.sync_copy(ones_hbm, ones_v)
        zoff = pl.multiple_of(sid * ZB, 8)

        def zloop(z, carry):
            pltpu.sync_copy(zbuf, acc_sh.at[pl.ds(zoff + z * CHUNK, CHUNK)])
            return carry

        lax.fori_loop(0, ZB // CHUNK, zloop, 0)
        plsc.subcore_barrier()
        tile_base = sid * (chunks * CHUNK)

        def body(k, carry):
            eb = pl.multiple_of(tile_base + k * CHUNK, 8)
            pltpu.sync_copy(dst_hbm.at[pl.ds(eb, CHUNK)], idx_v)
            for j in range(CHUNK // 16):
                d = idx_v[pl.ds(j * 16, 16)]
                loc = d - rebase
                ok = (loc >= 0) & (loc < HALF)
                idx_v[pl.ds(j * 16, 16)] = jnp.where(ok, loc, HALF)
            pltpu.sync_copy(ones_v, acc_sh.at[idx_v], add=True)
            return carry

        lax.fori_loop(0, chunks, body, 0)
        plsc.subcore_barrier()
        foff = sid * FLUSH

        def floop(f, carry):
            ro = foff + f * FCH
            pltpu.sync_copy(acc_sh.at[pl.ds(ro, FCH)], zbuf.at[pl.ds(0, FCH)])
            pltpu.sync_copy(zbuf.at[pl.ds(0, FCH)], c_hbm.at[pl.ds(rebase + ro, FCH)])
            return carry

        lax.fori_loop(0, FLUSH // FCH, floop, 0)

    return cnt


def _make_proj(k_dim):
    def body(x_ref, wt_ref, o_ref):
        o_ref[...] = lax.dot_general(
            x_ref[...], wt_ref[...], (((1,), (0,)), ((), ())),
            preferred_element_type=jnp.float32)

    return pl.pallas_call(
        body,
        grid=(N // BLK,),
        in_specs=[
            pl.BlockSpec((BLK, k_dim), lambda i: (i, 0)),
            pl.BlockSpec((k_dim, DM), lambda i: (0, 0)),
        ],
        out_specs=pl.BlockSpec((BLK, DM), lambda i: (i, 0)),
        out_shape=jax.ShapeDtypeStruct((N, DM), jnp.float32),
    )


def _make_combine(n_rel, k_dim, norm, scale):
    def body(*refs):
        o_ref = refs[-1]
        x_ref = refs[2 * n_rel]
        wt_ref = refs[2 * n_rel + 1]
        b_ref = refs[2 * n_rel + 2]
        u = jnp.zeros((BLK, DM), jnp.float32)
        for r in range(n_rel):
            s = refs[2 * r][...]
            c = refs[2 * r + 1][...]
            u = u + s / jnp.maximum(c, 1.0)
        out = (scale * u
               + lax.dot_general(x_ref[...], wt_ref[...],
                                 (((1,), (0,)), ((), ())),
                                 preferred_element_type=jnp.float32)
               + b_ref[0:1, :])
        if norm:
            nn = jnp.sqrt(jnp.sum(out * out, axis=1, keepdims=True))
            out = out / jnp.maximum(nn, 1e-12)
        else:
            out = jnp.maximum(out, 0.0)
        o_ref[...] = out

    in_specs = []
    for _ in range(n_rel):
        in_specs.append(pl.BlockSpec((BLK, DM), lambda i: (i, 0)))
        in_specs.append(pl.BlockSpec((BLK, DM), lambda i: (i, 0)))
    in_specs.append(pl.BlockSpec((BLK, k_dim), lambda i: (i, 0)))
    in_specs.append(pl.BlockSpec((k_dim, DM), lambda i: (0, 0)))
    in_specs.append(pl.BlockSpec((8, DM), lambda i: (0, 0)))

    return pl.pallas_call(
        body,
        grid=(N // BLK,),
        in_specs=in_specs,
        out_specs=pl.BlockSpec((BLK, DM), lambda i: (i, 0)),
        out_shape=jax.ShapeDtypeStruct((N, DM), jnp.float32),
    )


def kernel(x_circ, x_mir, x_dis, ei_c_int_m, ei_m_int_d, ei_c_as_d,
           ei_m_rev_c, ei_d_rev_m, ei_d_rva_c, ei_c_gip_c, ei_m_gip_m,
           W1l, W1r, b1, W2l, W2r, b2, W3l, W3r, b3):
    eis = [ei_c_int_m, ei_m_int_d, ei_c_as_d, ei_m_rev_c, ei_d_rev_m,
           ei_d_rva_c, ei_c_gip_c, ei_m_gip_m]
    e = eis[0].shape[1]
    e_pad = -(-e // (NS * BCH * 2)) * (NS * BCH * 2)
    pad = e_pad - e

    srcs, dsts = [], []
    for ei in eis:
        srcs.append(jnp.concatenate([ei[0], jnp.zeros((pad,), jnp.int32)]))
        dsts.append(jnp.concatenate([ei[1], jnp.full((pad,), N, jnp.int32)]))

    zeros_blk = jnp.zeros((CHUNK, DM), jnp.float32)
    ones_blk = jnp.ones((CHUNK, DM), jnp.float32)

    segsum = _make_segsum(e_pad)
    count = _make_count(e_pad)
    proj128 = _make_proj(128)
    proj64 = _make_proj(DM)

    cnts = [count(d, zeros_blk, ones_blk) for d in dsts]

    rel_src = [0, 1, 0, 1, 2, 2, 0, 1]
    rels12 = {0: [3, 5], 1: [0, 4], 2: [1, 2]}
    rels3 = {0: [3, 5, 6], 1: [0, 4, 7], 2: [1, 2]}

    comb2 = _make_combine(2, DM, False, 0.5)
    comb2_128 = _make_combine(2, 128, False, 0.5)
    comb3n = _make_combine(3, DM, True, 1.0 / 3.0)
    comb2n = _make_combine(2, DM, True, 0.5)

    def layer(xs, Wl, Wr, b, k_dim, rels_by_dst, norm):
        proj = proj128 if k_dim == 128 else proj64
        outs = []
        for t in (0, 1, 2):
            rr = rels_by_dst[t]
            scale = 1.0 / len(rr)
            ops = []
            for r in rr:
                p = proj(xs[rel_src[r]], jnp.transpose(Wl[r]))
                s = segsum(p, srcs[r], dsts[r], zeros_blk)
                ops.extend([s, cnts[r]])
            wt_sum = jnp.transpose(sum(Wr[r] for r in rr)) * scale
            b_sum = jnp.tile((sum(b[r] for r in rr) * scale)[None, :], (8, 1))
            if norm:
                comb = comb3n if len(rr) == 3 else comb2n
            else:
                comb = comb2_128 if k_dim == 128 else comb2
            outs.append(comb(*ops, xs[t], wt_sum, b_sum))
        return outs

    xs0 = [x_circ, x_mir, x_dis]
    h = layer(xs0, W1l, W1r, b1, 128, rels12, False)
    g = layer(h, W2l, W2r, b2, DM, rels12, False)
    o = layer(g, W3l, W3r, b3, DM, rels3, True)
    return o[0], o[1], o[2]


# speedup vs baseline: 2.0887x; 1.0007x over previous
import functools

import jax
import jax.numpy as jnp
from jax import lax
from jax.experimental import pallas as pl
from jax.experimental.pallas import tpu as pltpu
from jax.experimental.pallas import tpu_sc as plsc

N = 40000
DM = 64
HALF = N // 2
NS = 16
CHUNK = 128
BCH = 256
ZB = 1280
ACC_ROWS = ZB * NS
FLUSH = HALF // NS
FCH = 125
BLK = 2000


def _sc_mesh():
    return plsc.VectorSubcoreMesh(core_axis_name="c", subcore_axis_name="s")


def _make_segsum(e_pad):
    chunks = e_pad // (NS * BCH)
    nb = BCH // CHUNK

    def _buf_types():
        return [[pltpu.VMEM((CHUNK,), jnp.int32) for _ in range(nb)],
                pltpu.VMEM((BCH, DM), jnp.float32),
                pltpu.SemaphoreType.DMA]

    def _ibuf_types():
        return [pltpu.VMEM((BCH,), jnp.int32),
                pltpu.VMEM((BCH,), jnp.int32),
                pltpu.SemaphoreType.DMA]

    @functools.partial(
        pl.kernel,
        out_type=jax.ShapeDtypeStruct((N, DM), jnp.float32),
        mesh=_sc_mesh(),
        compiler_params=pltpu.CompilerParams(use_tc_tiling_on_sc=False),
        scratch_types=[
            pltpu.VMEM_SHARED((ACC_ROWS, DM), jnp.float32),
            pltpu.VMEM((CHUNK, DM), jnp.float32),
            _buf_types(),
            _buf_types(),
            [_ibuf_types() for _ in range(4)],
        ],
    )
    def seg(p_hbm, src_hbm, dst_hbm, zeros_hbm, s_hbm,
            acc_sh, zbuf, buf_a, buf_b, ibufs):
        cid = lax.axis_index("c")
        sid = lax.axis_index("s")
        rebase = cid * HALF
        pltpu.sync_copy(zeros_hbm, zbuf)
        zoff = pl.multiple_of(sid * ZB, 8)

        def zloop(z, carry):
            pltpu.sync_copy(zbuf, acc_sh.at[pl.ds(zoff + z * CHUNK, CHUNK)])
            return carry

        lax.fori_loop(0, ZB // CHUNK, zloop, 0)
        plsc.subcore_barrier()
        tile_base = sid * (chunks * BCH)

        def load_idx(ib, c):
            src_v, dst_v, sem = ib
            eb = pl.multiple_of(tile_base + c * BCH, 8)
            pltpu.async_copy(src_hbm.at[pl.ds(eb, BCH)], src_v, sem)
            pltpu.async_copy(dst_hbm.at[pl.ds(eb, BCH)], dst_v, sem)

        def load_idx_guarded(ib, c):
            @pl.when(c < chunks)
            def _():
                load_idx(ib, c)

        def issue(buf, ib, c):
            idxs, rows_v, gsem = buf
            src_v, dst_v, isem = ib
            eb = pl.multiple_of(tile_base + c * BCH, 8)
            pltpu.make_async_copy(src_hbm.at[pl.ds(eb, BCH)], src_v, isem).wait()
            pltpu.make_async_copy(dst_hbm.at[pl.ds(eb, BCH)], dst_v, isem).wait()
            for j in range(nb):
                pltpu.async_copy(p_hbm.at[src_v.at[pl.ds(j * CHUNK, CHUNK)]],
                                 rows_v.at[pl.ds(j * CHUNK, CHUNK)], gsem)
            for j in range(nb):
                for i in range(CHUNK // 16):
                    d = dst_v[pl.ds(j * CHUNK + i * 16, 16)]
                    loc = d - rebase
                    ok = (loc >= 0) & (loc < HALF)
                    idxs[j][pl.ds(i * 16, 16)] = jnp.where(ok, loc, HALF)

        def drain_scatter(buf, ib):
            idxs, rows_v, gsem = buf
            src_v, dst_v, isem = ib
            for j in range(nb):
                pltpu.make_async_copy(
                    p_hbm.at[src_v.at[pl.ds(j * CHUNK, CHUNK)]],
                    rows_v.at[pl.ds(j * CHUNK, CHUNK)], gsem).wait()
            for j in range(nb):
                pltpu.sync_copy(rows_v.at[pl.ds(j * CHUNK, CHUNK)],
                                acc_sh.at[idxs[j]], add=True)

        for b in range(4):
            load_idx(ibufs[b], b)
        issue(buf_a, ibufs[0], 0)

        def body(u, carry):
            c0 = 4 * u
            issue(buf_b, ibufs[1], c0 + 1)
            drain_scatter(buf_a, ibufs[0])
            load_idx_guarded(ibufs[0], c0 + 4)
            issue(buf_a, ibufs[2], c0 + 2)
            drain_scatter(buf_b, ibufs[1])
            load_idx_guarded(ibufs[1], c0 + 5)
            issue(buf_b, ibufs[3], c0 + 3)
            drain_scatter(buf_a, ibufs[2])
            load_idx_guarded(ibufs[2], c0 + 6)

            @pl.when(c0 + 4 < chunks)
            def _():
                issue(buf_a, ibufs[0], c0 + 4)

            drain_scatter(buf_b, ibufs[3])
            load_idx_guarded(ibufs[3], c0 + 7)
            return carry

        lax.fori_loop(0, chunks // 4, body, 0)
        plsc.subcore_barrier()
        foff = sid * FLUSH

        def floop(f, carry):
            ro = foff + f * FCH
            pltpu.sync_copy(acc_sh.at[pl.ds(ro, FCH)], zbuf.at[pl.ds(0, FCH)])
            pltpu.sync_copy(zbuf.at[pl.ds(0, FCH)], s_hbm.at[pl.ds(rebase + ro, FCH)])
            return carry

        lax.fori_loop(0, FLUSH // FCH, floop, 0)

    return seg


def _make_count(e_pad):
    chunks = e_pad // (NS * CHUNK)

    @functools.partial(
        pl.kernel,
        out_type=jax.ShapeDtypeStruct((N, DM), jnp.float32),
        mesh=_sc_mesh(),
        compiler_params=pltpu.CompilerParams(use_tc_tiling_on_sc=False),
        scratch_types=[
            pltpu.VMEM_SHARED((ACC_ROWS, DM), jnp.float32),
            pltpu.VMEM((CHUNK,), jnp.int32),
            pltpu.VMEM((CHUNK, DM), jnp.float32),
            pltpu.VMEM((CHUNK, DM), jnp.float32),
        ],
    )
    def cnt(dst_hbm, zeros_hbm, ones_hbm, c_hbm, acc_sh, idx_v, ones_v, zbuf):
        cid = lax.axis_index("c")
        sid = lax.axis_index("s")
        rebase = cid * HALF
        pltpu.sync_copy(zeros_hbm, zbuf)
        pltpu.sync_copy(ones_hbm, ones_v)
        zoff = pl.multiple_of(sid * ZB, 8)

        def zloop(z, carry):
            pltpu.sync_copy(zbuf, acc_sh.at[pl.ds(zoff + z * CHUNK, CHUNK)])
            return carry

        lax.fori_loop(0, ZB // CHUNK, zloop, 0)
        plsc.subcore_barrier()
        tile_base = sid * (chunks * CHUNK)

        def body(k, carry):
            eb = pl.multiple_of(tile_base + k * CHUNK, 8)
            pltpu.sync_copy(dst_hbm.at[pl.ds(eb, CHUNK)], idx_v)
            for j in range(CHUNK // 16):
                d = idx_v[pl.ds(j * 16, 16)]
                loc = d - rebase
                ok = (loc >= 0) & (loc < HALF)
                idx_v[pl.ds(j * 16, 16)] = jnp.where(ok, loc, HALF)
            pltpu.sync_copy(ones_v, acc_sh.at[idx_v], add=True)
            return carry

        lax.fori_loop(0, chunks, body, 0)
        plsc.subcore_barrier()
        foff = sid * FLUSH

        def floop(f, carry):
            ro = foff + f * FCH
            pltpu.sync_copy(acc_sh.at[pl.ds(ro, FCH)], zbuf.at[pl.ds(0, FCH)])
            pltpu.sync_copy(zbuf.at[pl.ds(0, FCH)], c_hbm.at[pl.ds(rebase + ro, FCH)])
            return carry

        lax.fori_loop(0, FLUSH // FCH, floop, 0)

    return cnt


def _make_proj(k_dim):
    def body(x_ref, wt_ref, o_ref):
        o_ref[...] = lax.dot_general(
            x_ref[...], wt_ref[...], (((1,), (0,)), ((), ())),
            preferred_element_type=jnp.float32)

    return pl.pallas_call(
        body,
        grid=(N // BLK,),
        in_specs=[
            pl.BlockSpec((BLK, k_dim), lambda i: (i, 0)),
            pl.BlockSpec((k_dim, DM), lambda i: (0, 0)),
        ],
        out_specs=pl.BlockSpec((BLK, DM), lambda i: (i, 0)),
        out_shape=jax.ShapeDtypeStruct((N, DM), jnp.float32),
    )


def _make_combine(n_rel, k_dim, norm, scale):
    def body(*refs):
        o_ref = refs[-1]
        x_ref = refs[2 * n_rel]
        wt_ref = refs[2 * n_rel + 1]
        b_ref = refs[2 * n_rel + 2]
        u = jnp.zeros((BLK, DM), jnp.float32)
        for r in range(n_rel):
            s = refs[2 * r][...]
            c = refs[2 * r + 1][...]
            u = u + s / jnp.maximum(c, 1.0)
        out = (scale * u
               + lax.dot_general(x_ref[...], wt_ref[...],
                                 (((1,), (0,)), ((), ())),
                                 preferred_element_type=jnp.float32)
               + b_ref[0:1, :])
        if norm:
            nn = jnp.sqrt(jnp.sum(out * out, axis=1, keepdims=True))
            out = out / jnp.maximum(nn, 1e-12)
        else:
            out = jnp.maximum(out, 0.0)
        o_ref[...] = out

    in_specs = []
    for _ in range(n_rel):
        in_specs.append(pl.BlockSpec((BLK, DM), lambda i: (i, 0)))
        in_specs.append(pl.BlockSpec((BLK, DM), lambda i: (i, 0)))
    in_specs.append(pl.BlockSpec((BLK, k_dim), lambda i: (i, 0)))
    in_specs.append(pl.BlockSpec((k_dim, DM), lambda i: (0, 0)))
    in_specs.append(pl.BlockSpec((8, DM), lambda i: (0, 0)))

    return pl.pallas_call(
        body,
        grid=(N // BLK,),
        in_specs=in_specs,
        out_specs=pl.BlockSpec((BLK, DM), lambda i: (i, 0)),
        out_shape=jax.ShapeDtypeStruct((N, DM), jnp.float32),
    )


def kernel(x_circ, x_mir, x_dis, ei_c_int_m, ei_m_int_d, ei_c_as_d,
           ei_m_rev_c, ei_d_rev_m, ei_d_rva_c, ei_c_gip_c, ei_m_gip_m,
           W1l, W1r, b1, W2l, W2r, b2, W3l, W3r, b3):
    eis = [ei_c_int_m, ei_m_int_d, ei_c_as_d, ei_m_rev_c, ei_d_rev_m,
           ei_d_rva_c, ei_c_gip_c, ei_m_gip_m]
    e = eis[0].shape[1]
    e_pad = -(-e // (NS * BCH * 4)) * (NS * BCH * 4)
    pad = e_pad - e

    srcs, dsts = [], []
    for ei in eis:
        srcs.append(jnp.concatenate([ei[0], jnp.zeros((pad,), jnp.int32)]))
        dsts.append(jnp.concatenate([ei[1], jnp.full((pad,), N, jnp.int32)]))

    zeros_blk = jnp.zeros((CHUNK, DM), jnp.float32)
    ones_blk = jnp.ones((CHUNK, DM), jnp.float32)

    segsum = _make_segsum(e_pad)
    count = _make_count(e_pad)
    proj128 = _make_proj(128)
    proj64 = _make_proj(DM)

    cnts = [count(d, zeros_blk, ones_blk) for d in dsts]

    rel_src = [0, 1, 0, 1, 2, 2, 0, 1]
    rels12 = {0: [3, 5], 1: [0, 4], 2: [1, 2]}
    rels3 = {0: [3, 5, 6], 1: [0, 4, 7], 2: [1, 2]}

    comb2 = _make_combine(2, DM, False, 0.5)
    comb2_128 = _make_combine(2, 128, False, 0.5)
    comb3n = _make_combine(3, DM, True, 1.0 / 3.0)
    comb2n = _make_combine(2, DM, True, 0.5)

    def layer(xs, Wl, Wr, b, k_dim, rels_by_dst, norm):
        proj = proj128 if k_dim == 128 else proj64
        outs = []
        for t in (0, 1, 2):
            rr = rels_by_dst[t]
            scale = 1.0 / len(rr)
            ops = []
            for r in rr:
                p = proj(xs[rel_src[r]], jnp.transpose(Wl[r]))
                s = segsum(p, srcs[r], dsts[r], zeros_blk)
                ops.extend([s, cnts[r]])
            wt_sum = jnp.transpose(sum(Wr[r] for r in rr)) * scale
            b_sum = jnp.tile((sum(b[r] for r in rr) * scale)[None, :], (8, 1))
            if norm:
                comb = comb3n if len(rr) == 3 else comb2n
            else:
                comb = comb2_128 if k_dim == 128 else comb2
            outs.append(comb(*ops, xs[t], wt_sum, b_sum))
        return outs

    xs0 = [x_circ, x_mir, x_dis]
    h = layer(xs0, W1l, W1r, b1, 128, rels12, False)
    g = layer(h, W2l, W2r, b2, DM, rels12, False)
    o = layer(g, W3l, W3r, b3, DM, rels3, True)
    return o[0], o[1], o[2]
